# baseline (speedup 1.0000x reference)
import functools

import jax
import jax.numpy as jnp
from jax import lax
from jax.experimental import pallas as pl
from jax.experimental.pallas import tpu as pltpu
from jax.experimental.pallas import tpu_sc as plsc

N = 10000
E = 320000
D = 128
G = 64
NC = 2
NS = 16
NW = NC * NS
CHUNK = 64
NPASS = 2
NHALF = 80
NCHUNK = NPASS * NHALF
EPW = NCHUNK * CHUNK
EPAD = NW * EPW
RPW = 632
NPAD = NS * RPW


@functools.cache
def _build_sc_propagate():
    mesh = plsc.VectorSubcoreMesh(core_axis_name="c", subcore_axis_name="s")

    @functools.partial(
        pl.kernel,
        out_type=jax.ShapeDtypeStruct((NC, NPAD, D), jnp.float32),
        mesh=mesh,
        scratch_types=[
            pltpu.VMEM((NHALF, CHUNK), jnp.int32),
            pltpu.VMEM((NHALF, CHUNK), jnp.int32),
            pltpu.VMEM((CHUNK, D), jnp.float32),
            pltpu.VMEM((CHUNK, D), jnp.float32),
            pltpu.VMEM_SHARED((NPAD, D), jnp.float32),
            pltpu.SemaphoreType.DMA,
            pltpu.SemaphoreType.DMA,
        ],
    )
    def sc_propagate(m_hbm, src_hbm, dst_hbm, zeros_hbm, out_hbm,
                     src_v, dst_v, rows0, rows1, acc, g0, g1):
        c = lax.axis_index("c")
        s = lax.axis_index("s")
        wid = c * NS + s

        pltpu.sync_copy(zeros_hbm, acc.at[pl.ds(s * RPW, RPW)])
        plsc.subcore_barrier()

        for p in range(NPASS):
            pltpu.sync_copy(src_hbm.at[wid].at[p], src_v)
            pltpu.sync_copy(dst_hbm.at[wid].at[p], dst_v)

            pltpu.async_copy(m_hbm.at[src_v.at[0]], rows0, g0)

            @pl.loop(0, NHALF // 2)
            def _(i):
                j = 2 * i
                pltpu.make_async_copy(m_hbm.at[src_v.at[j]], rows0, g0).wait()
                pltpu.async_copy(m_hbm.at[src_v.at[j + 1]], rows1, g1)
                pltpu.sync_copy(rows0, acc.at[dst_v.at[j]], add=True)
                pltpu.make_async_copy(m_hbm.at[src_v.at[j]], rows1, g1).wait()

                @pl.when(i < NHALF // 2 - 1)
                def _():
                    pltpu.async_copy(m_hbm.at[src_v.at[j + 2]], rows0, g0)

                pltpu.sync_copy(rows1, acc.at[dst_v.at[j + 1]], add=True)

        plsc.subcore_barrier()
        pltpu.sync_copy(acc.at[pl.ds(s * RPW, RPW)],
                        out_hbm.at[c].at[pl.ds(s * RPW, RPW)])

    return sc_propagate


def _sc_propagate(m, src, dst, zeros):
    return _build_sc_propagate()(m, src, dst, zeros)


ROWS_BLK = 2000


def _mm_body(h_ref, w_ref, o_ref):
    o_ref[...] = jnp.dot(h_ref[...], w_ref[...],
                         preferred_element_type=jnp.float32)


_mm = pl.pallas_call(
    _mm_body,
    grid=(N // ROWS_BLK,),
    in_specs=[
        pl.BlockSpec((ROWS_BLK, D), lambda i: (i, 0)),
        pl.BlockSpec((D, D), lambda i: (0, 0)),
    ],
    out_specs=pl.BlockSpec((ROWS_BLK, D), lambda i: (i, 0)),
    out_shape=jax.ShapeDtypeStruct((N, D), jnp.float32),
)


def _gru_body(a_ref, h_ref, wihT_ref, whhT_ref, bih_ref, bhh_ref, o_ref):
    a = a_ref[0] + a_ref[1]
    h = h_ref[...]
    gi = jnp.dot(a, wihT_ref[...], preferred_element_type=jnp.float32)
    gi = gi + bih_ref[...]
    gh = jnp.dot(h, whhT_ref[...], preferred_element_type=jnp.float32)
    gh = gh + bhh_ref[...]
    i_r, i_z, i_n = gi[:, :D], gi[:, D:2 * D], gi[:, 2 * D:]
    h_r, h_z, h_n = gh[:, :D], gh[:, D:2 * D], gh[:, 2 * D:]
    r = jax.nn.sigmoid(i_r + h_r)
    z = jax.nn.sigmoid(i_z + h_z)
    n = jnp.tanh(i_n + r * h_n)
    o_ref[...] = (1.0 - z) * n + z * h


_gru = pl.pallas_call(
    _gru_body,
    grid=(N // ROWS_BLK,),
    in_specs=[
        pl.BlockSpec((NC, ROWS_BLK, D), lambda i: (0, i, 0)),
        pl.BlockSpec((ROWS_BLK, D), lambda i: (i, 0)),
        pl.BlockSpec((D, 3 * D), lambda i: (0, 0)),
        pl.BlockSpec((D, 3 * D), lambda i: (0, 0)),
        pl.BlockSpec((1, 3 * D), lambda i: (0, 0)),
        pl.BlockSpec((1, 3 * D), lambda i: (0, 0)),
    ],
    out_specs=pl.BlockSpec((ROWS_BLK, D), lambda i: (i, 0)),
    out_shape=jax.ShapeDtypeStruct((N, D), jnp.float32),
)


def _final_body(h_ref, batch_ref, fc1T_ref, fc1b_ref, fc2Tp_ref, fc2bp_ref,
                o_ref):
    h = jax.nn.relu(h_ref[...])
    b = batch_ref[...]
    gids = lax.broadcasted_iota(jnp.int32, (1, G), 1).astype(jnp.float32)
    oh = (b == gids).astype(jnp.float32)
    sums = jax.lax.dot_general(oh, h, (((0,), (0,)), ((), ())),
                               preferred_element_type=jnp.float32)
    counts = jnp.sum(oh, axis=0)[:, None]
    pooled = sums / jnp.maximum(counts, 1.0)
    t = jax.nn.relu(
        jnp.dot(pooled, fc1T_ref[...], preferred_element_type=jnp.float32)
        + fc1b_ref[...])
    logits = jnp.dot(t, fc2Tp_ref[...], preferred_element_type=jnp.float32)
    logits = logits + fc2bp_ref[...]
    valid = lax.broadcasted_iota(jnp.int32, (G, D), 1) < 6
    neg = jnp.float32(-1e30)
    mx = jnp.max(jnp.where(valid, logits, neg), axis=1, keepdims=True)
    ex = jnp.where(valid, jnp.exp(logits - mx), 0.0)
    lse = jnp.log(jnp.sum(ex, axis=1, keepdims=True))
    o_ref[...] = logits - mx - lse


_final = pl.pallas_call(
    _final_body,
    in_specs=[
        pl.BlockSpec((N, D), lambda: (0, 0)),
        pl.BlockSpec((N, 1), lambda: (0, 0)),
        pl.BlockSpec((D, G), lambda: (0, 0)),
        pl.BlockSpec((1, G), lambda: (0, 0)),
        pl.BlockSpec((G, D), lambda: (0, 0)),
        pl.BlockSpec((1, D), lambda: (0, 0)),
    ],
    out_specs=pl.BlockSpec((G, D), lambda: (0, 0)),
    out_shape=jax.ShapeDtypeStruct((G, D), jnp.float32),
)


def kernel(x, edge_index, batch, Wg, W_ih, W_hh, b_ih, b_hh,
           fc1_w, fc1_b, fc2_w, fc2_b):
    pad = EPAD - E
    src = jnp.concatenate(
        [edge_index[0].astype(jnp.int32), jnp.zeros((pad,), jnp.int32)]
    ).reshape(NW, NPASS, NHALF, CHUNK)
    pad_dst = N + jnp.arange(pad, dtype=jnp.int32) % (NPAD - N)
    dst = jnp.concatenate(
        [edge_index[1].astype(jnp.int32), pad_dst]
    ).reshape(NW, NPASS, NHALF, CHUNK)
    zeros = jnp.zeros((RPW, D), jnp.float32)
    wihT = W_ih.T
    whhT = W_hh.T
    bih = b_ih.reshape(1, 3 * D)
    bhh = b_hh.reshape(1, 3 * D)
    batchf = batch.astype(jnp.float32).reshape(N, 1)
    fc1T = fc1_w.T
    fc1b = fc1_b.reshape(1, G)
    fc2Tp = jnp.zeros((G, D), jnp.float32).at[:, :6].set(fc2_w.T)
    fc2bp = jnp.zeros((1, D), jnp.float32).at[:, :6].set(fc2_b)

    h = x
    for i in range(2):
        m = _mm(h, Wg[i])
        parts = _sc_propagate(m, src, dst, zeros)
        h = _gru(parts, h, wihT, whhT, bih, bhh)
    out = _final(h, batchf, fc1T, fc1b, fc2Tp, fc2bp)
    return out[:, :6]


# speedup vs baseline: 2.4685x; 2.4685x over previous
import functools

import jax
import jax.numpy as jnp
from jax import lax
from jax.experimental import pallas as pl
from jax.experimental.pallas import tpu as pltpu
from jax.experimental.pallas import tpu_sc as plsc

N = 10000
E = 320000
D = 128
G = 64
NC = 2
NS = 16
NW = NC * NS
CHUNK = 125
NCHUNK = 80
EPW = NCHUNK * CHUNK
EPAD = NW * EPW
RPW = 632
NPAD = NS * RPW


@functools.cache
def _build_sc_propagate():
    mesh = plsc.VectorSubcoreMesh(core_axis_name="c", subcore_axis_name="s")

    @functools.partial(
        pl.kernel,
        out_type=jax.ShapeDtypeStruct((NC, NPAD, D), jnp.float32),
        mesh=mesh,
        scratch_types=[
            pltpu.VMEM((NCHUNK, CHUNK), jnp.int32),
            pltpu.VMEM((NCHUNK, CHUNK), jnp.int32),
            pltpu.VMEM((CHUNK, D), jnp.float32),
            pltpu.VMEM_SHARED((NPAD, D), jnp.float32),
        ],
    )
    def sc_propagate(m_hbm, src_hbm, dst_hbm, zeros_hbm, out_hbm,
                     src_v, dst_v, rows_v, acc):
        c = lax.axis_index("c")
        s = lax.axis_index("s")
        wid = c * NS + s

        pltpu.sync_copy(zeros_hbm, acc.at[pl.ds(s * RPW, RPW)])
        pltpu.sync_copy(src_hbm.at[wid], src_v)
        pltpu.sync_copy(dst_hbm.at[wid], dst_v)
        plsc.subcore_barrier()

        @pl.loop(0, NCHUNK)
        def _(j):
            pltpu.sync_copy(m_hbm.at[src_v.at[j]], rows_v)
            pltpu.sync_copy(rows_v, acc.at[dst_v.at[j]], add=True)

        plsc.subcore_barrier()
        pltpu.sync_copy(acc.at[pl.ds(s * RPW, RPW)],
                        out_hbm.at[c].at[pl.ds(s * RPW, RPW)])

    return sc_propagate


def _sc_propagate(m, src, dst, zeros):
    return _build_sc_propagate()(m, src, dst, zeros)


ROWS_BLK = 2000


def _mm_body(h_ref, w_ref, o_ref):
    o_ref[...] = jnp.dot(h_ref[...], w_ref[...],
                         preferred_element_type=jnp.float32)


_mm = pl.pallas_call(
    _mm_body,
    grid=(N // ROWS_BLK,),
    in_specs=[
        pl.BlockSpec((ROWS_BLK, D), lambda i: (i, 0)),
        pl.BlockSpec((D, D), lambda i: (0, 0)),
    ],
    out_specs=pl.BlockSpec((ROWS_BLK, D), lambda i: (i, 0)),
    out_shape=jax.ShapeDtypeStruct((N, D), jnp.float32),
)


def _gru_body(a_ref, h_ref, wihT_ref, whhT_ref, bih_ref, bhh_ref, o_ref):
    a = a_ref[0] + a_ref[1]
    h = h_ref[...]
    gi = jnp.dot(a, wihT_ref[...], preferred_element_type=jnp.float32)
    gi = gi + bih_ref[...]
    gh = jnp.dot(h, whhT_ref[...], preferred_element_type=jnp.float32)
    gh = gh + bhh_ref[...]
    i_r, i_z, i_n = gi[:, :D], gi[:, D:2 * D], gi[:, 2 * D:]
    h_r, h_z, h_n = gh[:, :D], gh[:, D:2 * D], gh[:, 2 * D:]
    r = jax.nn.sigmoid(i_r + h_r)
    z = jax.nn.sigmoid(i_z + h_z)
    n = jnp.tanh(i_n + r * h_n)
    o_ref[...] = (1.0 - z) * n + z * h


_gru = pl.pallas_call(
    _gru_body,
    grid=(N // ROWS_BLK,),
    in_specs=[
        pl.BlockSpec((NC, ROWS_BLK, D), lambda i: (0, i, 0)),
        pl.BlockSpec((ROWS_BLK, D), lambda i: (i, 0)),
        pl.BlockSpec((D, 3 * D), lambda i: (0, 0)),
        pl.BlockSpec((D, 3 * D), lambda i: (0, 0)),
        pl.BlockSpec((1, 3 * D), lambda i: (0, 0)),
        pl.BlockSpec((1, 3 * D), lambda i: (0, 0)),
    ],
    out_specs=pl.BlockSpec((ROWS_BLK, D), lambda i: (i, 0)),
    out_shape=jax.ShapeDtypeStruct((N, D), jnp.float32),
)


def _final_body(h_ref, batch_ref, fc1T_ref, fc1b_ref, fc2Tp_ref, fc2bp_ref,
                o_ref):
    h = jax.nn.relu(h_ref[...])
    b = batch_ref[...]
    gids = lax.broadcasted_iota(jnp.int32, (1, G), 1).astype(jnp.float32)
    oh = (b == gids).astype(jnp.float32)
    sums = jax.lax.dot_general(oh, h, (((0,), (0,)), ((), ())),
                               preferred_element_type=jnp.float32)
    counts = jnp.sum(oh, axis=0)[:, None]
    pooled = sums / jnp.maximum(counts, 1.0)
    t = jax.nn.relu(
        jnp.dot(pooled, fc1T_ref[...], preferred_element_type=jnp.float32)
        + fc1b_ref[...])
    logits = jnp.dot(t, fc2Tp_ref[...], preferred_element_type=jnp.float32)
    logits = logits + fc2bp_ref[...]
    valid = lax.broadcasted_iota(jnp.int32, (G, D), 1) < 6
    neg = jnp.float32(-1e30)
    mx = jnp.max(jnp.where(valid, logits, neg), axis=1, keepdims=True)
    ex = jnp.where(valid, jnp.exp(logits - mx), 0.0)
    lse = jnp.log(jnp.sum(ex, axis=1, keepdims=True))
    o_ref[...] = logits - mx - lse


_final = pl.pallas_call(
    _final_body,
    in_specs=[
        pl.BlockSpec((N, D), lambda: (0, 0)),
        pl.BlockSpec((N, 1), lambda: (0, 0)),
        pl.BlockSpec((D, G), lambda: (0, 0)),
        pl.BlockSpec((1, G), lambda: (0, 0)),
        pl.BlockSpec((G, D), lambda: (0, 0)),
        pl.BlockSpec((1, D), lambda: (0, 0)),
    ],
    out_specs=pl.BlockSpec((G, D), lambda: (0, 0)),
    out_shape=jax.ShapeDtypeStruct((G, D), jnp.float32),
)


def kernel(x, edge_index, batch, Wg, W_ih, W_hh, b_ih, b_hh,
           fc1_w, fc1_b, fc2_w, fc2_b):
    src = edge_index[0].astype(jnp.int32).reshape(NW, NCHUNK, CHUNK)
    dst = edge_index[1].astype(jnp.int32).reshape(NW, NCHUNK, CHUNK)
    zeros = jnp.zeros((RPW, D), jnp.float32)
    wihT = W_ih.T
    whhT = W_hh.T
    bih = b_ih.reshape(1, 3 * D)
    bhh = b_hh.reshape(1, 3 * D)
    batchf = batch.astype(jnp.float32).reshape(N, 1)
    fc1T = fc1_w.T
    fc1b = fc1_b.reshape(1, G)
    fc2Tp = jnp.zeros((G, D), jnp.float32).at[:, :6].set(fc2_w.T)
    fc2bp = jnp.zeros((1, D), jnp.float32).at[:, :6].set(fc2_b)

    h = x
    for i in range(2):
        m = _mm(h, Wg[i])
        parts = _sc_propagate(m, src, dst, zeros)
        h = _gru(parts, h, wihT, whhT, bih, bhh)
    out = _final(h, batchf, fc1T, fc1b, fc2Tp, fc2bp)
    return out[:, :6]
